# baseline (speedup 1.0000x reference)
import functools

import jax
import jax.numpy as jnp
from jax import lax
from jax.experimental import pallas as pl
from jax.experimental.pallas import tpu as pltpu
from jax.experimental.pallas import tpu_sc as plsc

_NC = 2
_NS = 16
_NW = _NC * _NS
_LANES = 16
_B = 4
_L = 256
_CH = 96
_STRIDE = 97
_PADW = 272
_ROWS_W = (_B * _L) // _NW
_ROW_ELEMS = _L * _CH


def _sc_generate(pseq_flat, legal_flat):
    mesh = plsc.VectorSubcoreMesh(
        core_axis_name="c", subcore_axis_name="s",
        num_cores=_NC, num_subcores=_NS)

    @functools.partial(
        pl.kernel,
        out_type=jax.ShapeDtypeStruct((_B * _L * _ROW_ELEMS,), jnp.float32),
        mesh=mesh,
        scratch_types=[
            pltpu.VMEM((_B * _PADW,), jnp.int32),
            pltpu.VMEM((_ROWS_W * _L,), jnp.float32),
            pltpu.VMEM((_L * _STRIDE,), jnp.float32),
            pltpu.VMEM((_ROW_ELEMS,), jnp.float32),
            pltpu.VMEM((_ROW_ELEMS,), jnp.float32),
            pltpu.SemaphoreType.DMA,
            pltpu.SemaphoreType.DMA,
        ],
        compiler_params=pltpu.CompilerParams(needs_layout_passes=False),
    )
    def gen(pseq_hbm, legal_hbm, out_hbm, pseq_v, legal_v, stage_v,
            row0_v, row1_v, sem0, sem1):
        wid = lax.axis_index("s") * _NC + lax.axis_index("c")
        b = wid // (_L // _ROWS_W)
        i0 = (wid % (_L // _ROWS_W)) * _ROWS_W

        pltpu.sync_copy(pseq_hbm, pseq_v)
        pltpu.sync_copy(
            legal_hbm.at[pl.ds(wid * _ROWS_W * _L, _ROWS_W * _L)], legal_v)

        iota = lax.iota(jnp.int32, _LANES)
        one = jnp.full((_LANES,), 1.0, jnp.float32)
        zero = jnp.full((_LANES,), 0.0, jnp.float32)
        bufs = ((row0_v, sem0), (row1_v, sem1))

        def fill_row(t):
            i = i0 + t
            pbase = b * _PADW + i
            ri = [plsc.load_gather(
                pseq_v, [jnp.full((_LANES,), pbase + k, jnp.int32)])
                for k in range(11)]

            @pl.loop(0, _L // _LANES, unroll=2)
            def _jgroup(jg):
                jbase = jg * _LANES
                jvec = iota + jbase
                jidx = jvec * _STRIDE
                legal_vec = legal_v[pl.ds(t * _L + jbase, _LANES)]

                cj5 = plsc.load_gather(
                    pseq_v, [iota + (b * _PADW + jbase + 5)])
                for d in range(4):
                    plsc.store_scatter(
                        stage_v, [jidx + d], jnp.where(ri[5] == d, one, zero))
                    plsc.store_scatter(
                        stage_v, [jidx + (4 + d)],
                        jnp.where(cj5 == d, one, zero))

                for k in range(11):
                    cj = cj5 if k == 5 else plsc.load_gather(
                        pseq_v, [iota + (b * _PADW + jbase + 10 - k)])
                    canon = ((ri[k] + cj) == 3) | ((ri[k] * cj) == 6)
                    m = canon & (jvec > (i + 2 * k - 7))
                    w = jnp.where(m, legal_vec, zero)
                    ch = 8 + 8 * k
                    for d in range(4):
                        plsc.store_scatter(
                            stage_v, [jidx + (ch + d)],
                            jnp.where(ri[k] == d, w, zero))
                        plsc.store_scatter(
                            stage_v, [jidx + (ch + 4 + d)],
                            jnp.where(cj == d, w, zero))

        def repack(buf):
            @pl.loop(0, _L, unroll=4)
            def _j(j):
                src = j * _STRIDE
                dst = j * _CH
                for tt in range(_CH // _LANES):
                    v = plsc.load_gather(
                        stage_v, [iota + (src + _LANES * tt)])
                    buf[pl.ds(dst + _LANES * tt, _LANES)] = v

        @pl.loop(0, _ROWS_W, step=2)
        def _rowpair(t2):
            for p, (buf, sem) in enumerate(bufs):
                t = t2 + p
                fill_row(t)

                @pl.when(t2 > 0)
                def _wait_prev():
                    pltpu.make_async_copy(
                        buf, out_hbm.at[pl.ds(0, _ROW_ELEMS)], sem).wait()

                repack(buf)
                pltpu.async_copy(
                    buf,
                    out_hbm.at[pl.ds((b * _L + i0 + t) * _ROW_ELEMS,
                                     _ROW_ELEMS)],
                    sem)

        for buf, sem in bufs:
            pltpu.make_async_copy(
                buf, out_hbm.at[pl.ds(0, _ROW_ELEMS)], sem).wait()

    return gen(pseq_flat, legal_flat)


def kernel(seq_indices, legal_mask):
    B, L = seq_indices.shape
    pseq = jnp.full((B, _PADW), 4, jnp.int32)
    pseq = pseq.at[:, 5:5 + L].set(seq_indices.astype(jnp.int32))
    out_flat = _sc_generate(pseq.reshape(-1), legal_mask.reshape(-1))
    return out_flat.reshape(B, L, L, _CH)


# speedup vs baseline: 1.4155x; 1.4155x over previous
import functools

import jax
import jax.numpy as jnp
from jax import lax
from jax.experimental import pallas as pl
from jax.experimental.pallas import tpu as pltpu
from jax.experimental.pallas import tpu_sc as plsc

_NC = 2
_NS = 16
_NW = _NC * _NS
_LANES = 16
_B = 4
_L = 256
_CH = 96
_STRIDE = 97
_PADW = 272
_ROWS_W = (_B * _L) // _NW
_ROW_ELEMS = _L * _CH


def _sc_generate(pseq_flat, legal_flat):
    mesh = plsc.VectorSubcoreMesh(
        core_axis_name="c", subcore_axis_name="s",
        num_cores=_NC, num_subcores=_NS)

    @functools.partial(
        pl.kernel,
        out_type=jax.ShapeDtypeStruct((_B * _L * _ROW_ELEMS,), jnp.float32),
        mesh=mesh,
        scratch_types=[
            pltpu.VMEM((_B * _PADW,), jnp.int32),
            pltpu.VMEM((_ROWS_W * _L,), jnp.float32),
            pltpu.VMEM((_L * _STRIDE,), jnp.float32),
            pltpu.VMEM((_ROW_ELEMS,), jnp.float32),
            pltpu.VMEM((_ROW_ELEMS,), jnp.float32),
            pltpu.SemaphoreType.DMA,
            pltpu.SemaphoreType.DMA,
        ],
        compiler_params=pltpu.CompilerParams(needs_layout_passes=False),
    )
    def gen(pseq_hbm, legal_hbm, out_hbm, pseq_v, legal_v, stage_v,
            row0_v, row1_v, sem0, sem1):
        wid = lax.axis_index("s") * _NC + lax.axis_index("c")
        b = wid // (_L // _ROWS_W)
        i0 = (wid % (_L // _ROWS_W)) * _ROWS_W

        pltpu.sync_copy(pseq_hbm, pseq_v)
        pltpu.sync_copy(
            legal_hbm.at[pl.ds(wid * _ROWS_W * _L, _ROWS_W * _L)], legal_v)

        iota = lax.iota(jnp.int32, _LANES)
        one = jnp.full((_LANES,), 1.0, jnp.float32)
        zero = jnp.full((_LANES,), 0.0, jnp.float32)
        bufs = ((row0_v, sem0), (row1_v, sem1))

        def fill_row(t):
            i = i0 + t
            pbase = b * _PADW + i
            ri = [plsc.load_gather(
                pseq_v, [jnp.full((_LANES,), pbase + k, jnp.int32)])
                for k in range(11)]

            @plsc.parallel_loop(0, _L // _LANES, unroll=2)
            def _jgroup(jg):
                jbase = jg * _LANES
                jvec = iota + jbase
                jidx = jvec * _STRIDE
                legal_vec = legal_v[pl.ds(t * _L + jbase, _LANES)]

                cj5 = plsc.load_gather(
                    pseq_v, [iota + (b * _PADW + jbase + 5)])
                for d in range(4):
                    plsc.store_scatter(
                        stage_v, [jidx + d], jnp.where(ri[5] == d, one, zero))
                    plsc.store_scatter(
                        stage_v, [jidx + (4 + d)],
                        jnp.where(cj5 == d, one, zero))

                for k in range(11):
                    cj = cj5 if k == 5 else plsc.load_gather(
                        pseq_v, [iota + (b * _PADW + jbase + 10 - k)])
                    canon = ((ri[k] + cj) == 3) | ((ri[k] * cj) == 6)
                    m = canon & (jvec > (i + 2 * k - 7))
                    w = jnp.where(m, legal_vec, zero)
                    ch = 8 + 8 * k
                    for d in range(4):
                        plsc.store_scatter(
                            stage_v, [jidx + (ch + d)],
                            jnp.where(ri[k] == d, w, zero))
                        plsc.store_scatter(
                            stage_v, [jidx + (ch + 4 + d)],
                            jnp.where(cj == d, w, zero))

        def repack(buf):
            @plsc.parallel_loop(0, _L, unroll=4)
            def _j(j):
                src = j * _STRIDE
                dst = j * _CH
                for tt in range(_CH // _LANES):
                    v = plsc.load_gather(
                        stage_v, [iota + (src + _LANES * tt)])
                    buf[pl.ds(dst + _LANES * tt, _LANES)] = v

        @pl.loop(0, _ROWS_W, step=2)
        def _rowpair(t2):
            for p, (buf, sem) in enumerate(bufs):
                t = t2 + p
                fill_row(t)

                @pl.when(t2 > 0)
                def _wait_prev():
                    pltpu.make_async_copy(
                        buf, out_hbm.at[pl.ds(0, _ROW_ELEMS)], sem).wait()

                repack(buf)
                pltpu.async_copy(
                    buf,
                    out_hbm.at[pl.ds((b * _L + i0 + t) * _ROW_ELEMS,
                                     _ROW_ELEMS)],
                    sem)

        for buf, sem in bufs:
            pltpu.make_async_copy(
                buf, out_hbm.at[pl.ds(0, _ROW_ELEMS)], sem).wait()

    return gen(pseq_flat, legal_flat)


def kernel(seq_indices, legal_mask):
    B, L = seq_indices.shape
    pseq = jnp.full((B, _PADW), 4, jnp.int32)
    pseq = pseq.at[:, 5:5 + L].set(seq_indices.astype(jnp.int32))
    out_flat = _sc_generate(pseq.reshape(-1), legal_mask.reshape(-1))
    return out_flat.reshape(B, L, L, _CH)


# speedup vs baseline: 1.9479x; 1.3761x over previous
import functools

import jax
import jax.numpy as jnp
from jax import lax
from jax.experimental import pallas as pl
from jax.experimental.pallas import tpu as pltpu
from jax.experimental.pallas import tpu_sc as plsc

_NC = 2
_NS = 16
_NW = _NC * _NS
_LANES = 16
_B = 4
_L = 256
_CH = 96
_STRIDE = 97
_PADW = 272
_ROWS_W = (_B * _L) // _NW
_ROW_ELEMS = _L * _CH


def _sc_generate(pseq_flat, legal_flat):
    mesh = plsc.VectorSubcoreMesh(
        core_axis_name="c", subcore_axis_name="s",
        num_cores=_NC, num_subcores=_NS)

    @functools.partial(
        pl.kernel,
        out_type=jax.ShapeDtypeStruct((_B, _L, _L, _CH), jnp.float32),
        mesh=mesh,
        scratch_types=[
            pltpu.VMEM((_B * _PADW,), jnp.int32),
            pltpu.VMEM((_ROWS_W * _L,), jnp.float32),
            pltpu.VMEM((_L * _STRIDE,), jnp.float32),
            pltpu.VMEM((_L, _CH), jnp.float32),
            pltpu.VMEM((_L, _CH), jnp.float32),
            pltpu.SemaphoreType.DMA,
            pltpu.SemaphoreType.DMA,
        ],
        compiler_params=pltpu.CompilerParams(needs_layout_passes=False),
    )
    def gen(pseq_hbm, legal_hbm, out_hbm, pseq_v, legal_v, stage_v,
            row0_v, row1_v, sem0, sem1):
        wid = lax.axis_index("s") * _NC + lax.axis_index("c")
        b = wid // (_L // _ROWS_W)
        i0 = (wid % (_L // _ROWS_W)) * _ROWS_W

        pltpu.sync_copy(pseq_hbm, pseq_v)
        pltpu.sync_copy(
            legal_hbm.at[pl.ds(wid * _ROWS_W * _L, _ROWS_W * _L)], legal_v)

        iota = lax.iota(jnp.int32, _LANES)
        one = jnp.full((_LANES,), 1.0, jnp.float32)
        zero = jnp.full((_LANES,), 0.0, jnp.float32)
        bufs = ((row0_v, sem0), (row1_v, sem1))

        def fill_row(t):
            i = i0 + t
            pbase = b * _PADW + i
            ri = [plsc.load_gather(
                pseq_v, [jnp.full((_LANES,), pbase + k, jnp.int32)])
                for k in range(11)]

            @plsc.parallel_loop(0, _L // _LANES, unroll=2)
            def _jgroup(jg):
                jbase = jg * _LANES
                jvec = iota + jbase
                jidx = jvec * _STRIDE
                legal_vec = legal_v[pl.ds(t * _L + jbase, _LANES)]

                cj5 = plsc.load_gather(
                    pseq_v, [iota + (b * _PADW + jbase + 5)])
                for d in range(4):
                    plsc.store_scatter(
                        stage_v, [jidx + d], jnp.where(ri[5] == d, one, zero))
                    plsc.store_scatter(
                        stage_v, [jidx + (4 + d)],
                        jnp.where(cj5 == d, one, zero))

                for k in range(11):
                    cj = cj5 if k == 5 else plsc.load_gather(
                        pseq_v, [iota + (b * _PADW + jbase + 10 - k)])
                    canon = ((ri[k] + cj) == 3) | ((ri[k] * cj) == 6)
                    m = canon & (jvec > (i + 2 * k - 7))
                    w = jnp.where(m, legal_vec, zero)
                    ch = 8 + 8 * k
                    for d in range(4):
                        plsc.store_scatter(
                            stage_v, [jidx + (ch + d)],
                            jnp.where(ri[k] == d, w, zero))
                        plsc.store_scatter(
                            stage_v, [jidx + (ch + 4 + d)],
                            jnp.where(cj == d, w, zero))

        def repack(buf):
            @plsc.parallel_loop(0, _L, unroll=4)
            def _j(j):
                src = j * _STRIDE
                for tt in range(_CH // _LANES):
                    v = plsc.load_gather(
                        stage_v, [iota + (src + _LANES * tt)])
                    buf[j, pl.ds(_LANES * tt, _LANES)] = v

        @pl.loop(0, _ROWS_W, step=2)
        def _rowpair(t2):
            for p, (buf, sem) in enumerate(bufs):
                t = t2 + p
                fill_row(t)

                @pl.when(t2 > 0)
                def _wait_prev():
                    pltpu.make_async_copy(
                        buf, out_hbm.at[0, 0], sem).wait()

                repack(buf)
                pltpu.async_copy(buf, out_hbm.at[b, i0 + t], sem)

        for buf, sem in bufs:
            pltpu.make_async_copy(buf, out_hbm.at[0, 0], sem).wait()

    return gen(pseq_flat, legal_flat)


def kernel(seq_indices, legal_mask):
    B, L = seq_indices.shape
    pseq = jnp.full((B, _PADW), 4, jnp.int32)
    pseq = pseq.at[:, 5:5 + L].set(seq_indices.astype(jnp.int32))
    return _sc_generate(pseq.reshape(-1), legal_mask.reshape(-1))


# speedup vs baseline: 2.3983x; 1.2312x over previous
import functools

import jax
import jax.numpy as jnp
from jax import lax
from jax.experimental import pallas as pl
from jax.experimental.pallas import tpu as pltpu
from jax.experimental.pallas import tpu_sc as plsc

_NC = 2
_NS = 16
_NW = _NC * _NS
_LANES = 16
_B = 4
_L = 256
_CH = 96
_STRIDE = 97
_PADW = 272
_ROWS_W = (_B * _L) // _NW
_ROW_ELEMS = _L * _CH


def _sc_generate(pseq_flat, legal_flat):
    mesh = plsc.VectorSubcoreMesh(
        core_axis_name="c", subcore_axis_name="s",
        num_cores=_NC, num_subcores=_NS)

    @functools.partial(
        pl.kernel,
        out_type=jax.ShapeDtypeStruct((_B, _L, _L, _CH), jnp.float32),
        mesh=mesh,
        scratch_types=[
            pltpu.VMEM((_B * _PADW,), jnp.int32),
            pltpu.VMEM((_ROWS_W * _L,), jnp.float32),
            pltpu.VMEM((_L * _STRIDE,), jnp.float32),
            pltpu.VMEM((_L, _CH), jnp.float32),
            pltpu.VMEM((_L, _CH), jnp.float32),
            pltpu.SemaphoreType.DMA,
            pltpu.SemaphoreType.DMA,
        ],
        compiler_params=pltpu.CompilerParams(needs_layout_passes=False),
    )
    def gen(pseq_hbm, legal_hbm, out_hbm, pseq_v, legal_v, stage_v,
            row0_v, row1_v, sem0, sem1):
        wid = lax.axis_index("s") * _NC + lax.axis_index("c")
        b = wid // (_L // _ROWS_W)
        i0 = (wid % (_L // _ROWS_W)) * _ROWS_W

        pltpu.sync_copy(pseq_hbm, pseq_v)
        pltpu.sync_copy(
            legal_hbm.at[pl.ds(wid * _ROWS_W * _L, _ROWS_W * _L)], legal_v)

        iota = lax.iota(jnp.int32, _LANES)
        one = jnp.full((_LANES,), 1.0, jnp.float32)
        zero = jnp.full((_LANES,), 0.0, jnp.float32)
        bufs = ((row0_v, sem0), (row1_v, sem1))

        def fill_row(t):
            i = i0 + t
            pbase = b * _PADW + i
            ri = [plsc.load_gather(
                pseq_v, [jnp.full((_LANES,), pbase + k, jnp.int32)])
                for k in range(11)]

            @plsc.parallel_loop(0, _L // _LANES, unroll=4)
            def _jgroup(jg):
                jbase = jg * _LANES
                jvec = iota + jbase
                jidx = jvec * _STRIDE
                legal_vec = legal_v[pl.ds(t * _L + jbase, _LANES)]

                cj5 = pseq_v[pl.ds(b * _PADW + jbase + 5, _LANES)]
                for d in range(4):
                    plsc.store_scatter(
                        stage_v, [jidx + d], jnp.where(ri[5] == d, one, zero))
                    plsc.store_scatter(
                        stage_v, [jidx + (4 + d)],
                        jnp.where(cj5 == d, one, zero))

                for k in range(11):
                    cj = cj5 if k == 5 else (
                        pseq_v[pl.ds(b * _PADW + jbase + 10 - k, _LANES)])
                    canon = ((ri[k] + cj) == 3) | ((ri[k] * cj) == 6)
                    m = canon & (jvec > (i + 2 * k - 7))
                    w = jnp.where(m, legal_vec, zero)
                    ch = 8 + 8 * k
                    for d in range(4):
                        plsc.store_scatter(
                            stage_v, [jidx + (ch + d)],
                            jnp.where(ri[k] == d, w, zero))
                        plsc.store_scatter(
                            stage_v, [jidx + (ch + 4 + d)],
                            jnp.where(cj == d, w, zero))

        def repack(buf):
            @plsc.parallel_loop(0, _L, unroll=4)
            def _j(j):
                src = j * _STRIDE
                for tt in range(_CH // _LANES):
                    v = plsc.load_gather(
                        stage_v, [iota + (src + _LANES * tt)])
                    buf[j, pl.ds(_LANES * tt, _LANES)] = v

        @pl.loop(0, _ROWS_W, step=2)
        def _rowpair(t2):
            for p, (buf, sem) in enumerate(bufs):
                t = t2 + p
                fill_row(t)

                @pl.when(t2 > 0)
                def _wait_prev():
                    pltpu.make_async_copy(
                        buf, out_hbm.at[0, 0], sem).wait()

                repack(buf)
                pltpu.async_copy(buf, out_hbm.at[b, i0 + t], sem)

        for buf, sem in bufs:
            pltpu.make_async_copy(buf, out_hbm.at[0, 0], sem).wait()

    return gen(pseq_flat, legal_flat)


def kernel(seq_indices, legal_mask):
    B, L = seq_indices.shape
    pseq = jnp.full((B, _PADW), 4, jnp.int32)
    pseq = pseq.at[:, 5:5 + L].set(seq_indices.astype(jnp.int32))
    return _sc_generate(pseq.reshape(-1), legal_mask.reshape(-1))
